# baseline (speedup 1.0000x reference)
import functools
import math

import jax
import jax.numpy as jnp
import numpy as np
from jax import lax
from jax.experimental import pallas as pl
from jax.experimental.pallas import tpu as pltpu
from jax.experimental.pallas import tpu_sc as plsc

N_ATOMS = 10000
N_PAIRS = 320000
NF = 128
N_SENS = 20
N_BLOCKS = 2
N_ATOM_LAYERS = 3
DIST_HARD_MAX = 6.5

_NC = 2
_NS = 16
_NW = _NC * _NS
_K = 128
_PER_W = N_PAIRS // _NW
_NCH = _PER_W // _K
_TAIL = _PER_W - _NCH * _K


def _build_gather():
    mesh = plsc.VectorSubcoreMesh(core_axis_name="c", subcore_axis_name="s")

    @functools.partial(
        pl.kernel,
        mesh=mesh,
        out_type=jax.ShapeDtypeStruct((N_PAIRS, NF), jnp.float32),
        scratch_types=[
            pltpu.VMEM((_PER_W,), jnp.int32),
            pltpu.VMEM((_K, NF), jnp.float32),
            pltpu.VMEM((_K, NF), jnp.float32),
            pltpu.SemaphoreType.DMA,
            pltpu.SemaphoreType.DMA,
        ],
    )
    def gk(feats_hbm, second_hbm, out_hbm, sec_v, rows0_v, rows1_v, sem0, sem1):
        cid = lax.axis_index("c")
        sid = lax.axis_index("s")
        wid = cid * _NS + sid
        base_p = wid * _PER_W
        pltpu.sync_copy(second_hbm.at[pl.ds(base_p, _PER_W)], sec_v)

        def issue(g, rows_v, sem):
            pltpu.async_copy(feats_hbm.at[sec_v.at[pl.ds(g * _K, _K)]], rows_v, sem)

        def drain(rows_v, sem):
            pltpu.make_async_copy(feats_hbm.at[sec_v.at[pl.ds(0, _K)]], rows_v, sem).wait()

        issue(0, rows0_v, sem0)

        def body(h, _):
            g0 = h * 2
            issue(g0 + 1, rows1_v, sem1)
            drain(rows0_v, sem0)
            pltpu.sync_copy(rows0_v, out_hbm.at[pl.ds(base_p + g0 * _K, _K)])

            @pl.when(g0 + 2 < _NCH)
            def _():
                issue(g0 + 2, rows0_v, sem0)

            drain(rows1_v, sem1)
            pltpu.sync_copy(rows1_v, out_hbm.at[pl.ds(base_p + (g0 + 1) * _K, _K)])
            return 0

        lax.fori_loop(0, _NCH // 2, body, 0)
        tp = base_p + _NCH * _K
        pltpu.async_copy(
            feats_hbm.at[sec_v.at[pl.ds(_NCH * _K, _TAIL)]],
            rows0_v.at[pl.ds(0, _TAIL)], sem0).wait()
        pltpu.sync_copy(rows0_v.at[pl.ds(0, _TAIL)], out_hbm.at[pl.ds(tp, _TAIL)])

    return gk


_gather_call = _build_gather()


def _build_scatter():
    mesh = plsc.VectorSubcoreMesh(core_axis_name="c", subcore_axis_name="s")

    @functools.partial(
        pl.kernel,
        mesh=mesh,
        out_type=jax.ShapeDtypeStruct((_NC, N_ATOMS, NF), jnp.float32),
        scratch_types=[
            pltpu.VMEM((_K,), jnp.int32),
            pltpu.VMEM((_K,), jnp.int32),
            pltpu.VMEM((_K, NF), jnp.float32),
            pltpu.VMEM((_K, NF), jnp.float32),
            pltpu.VMEM((_TAIL,), jnp.int32),
            pltpu.VMEM((104, NF), jnp.float32),
            pltpu.VMEM_SHARED((N_ATOMS, NF), jnp.float32),
            pltpu.SemaphoreType.DMA,
            pltpu.SemaphoreType.DMA,
        ],
    )
    def sk(c_hbm, first_hbm, out_hbm, fst0_v, fst1_v, cr0_v, cr1_v, fstt_v,
           zbuf_v, acc_sh, sem0, sem1):
        cid = lax.axis_index("c")
        sid = lax.axis_index("s")
        wid = cid * _NS + sid

        def zfill(i, _):
            zbuf_v[i // 8, pl.ds((i % 8) * 16, 16)] = jnp.zeros((16,), jnp.float32)
            return 0
        lax.fori_loop(0, 104 * 8, zfill, 0)

        def zcopy(i, _):
            pltpu.sync_copy(zbuf_v, acc_sh.at[pl.ds(sid * 624 + i * 104, 104)])
            return 0
        lax.fori_loop(0, 6, zcopy, 0)

        @pl.when(sid == _NS - 1)
        def _():
            pltpu.sync_copy(zbuf_v.at[pl.ds(0, 16)], acc_sh.at[pl.ds(9984, 16)])

        plsc.subcore_barrier()

        base_p = wid * _PER_W

        def load(g, fst_v, cr_v, sem):
            p0 = base_p + g * _K
            pltpu.async_copy(first_hbm.at[pl.ds(p0, _K)], fst_v, sem)
            pltpu.async_copy(c_hbm.at[pl.ds(p0, _K)], cr_v, sem)

        def drain(fst_v, cr_v, sem):
            pltpu.make_async_copy(first_hbm.at[pl.ds(0, _K)], fst_v, sem).wait()
            pltpu.make_async_copy(c_hbm.at[pl.ds(0, _K)], cr_v, sem).wait()

        def scat(fst_v, cr_v):
            pltpu.sync_copy(cr_v, acc_sh.at[fst_v], add=True)

        load(0, fst0_v, cr0_v, sem0)

        def body(h, _):
            g0 = h * 2
            load(g0 + 1, fst1_v, cr1_v, sem1)
            drain(fst0_v, cr0_v, sem0)
            scat(fst0_v, cr0_v)

            @pl.when(g0 + 2 < _NCH)
            def _():
                load(g0 + 2, fst0_v, cr0_v, sem0)

            drain(fst1_v, cr1_v, sem1)
            scat(fst1_v, cr1_v)
            return 0

        lax.fori_loop(0, _NCH // 2, body, 0)

        tp = base_p + _NCH * _K
        pltpu.sync_copy(first_hbm.at[pl.ds(tp, _TAIL)], fstt_v)
        pltpu.async_copy(c_hbm.at[pl.ds(tp, _TAIL)],
                         cr0_v.at[pl.ds(0, _TAIL)], sem0).wait()
        pltpu.sync_copy(cr0_v.at[pl.ds(0, _TAIL)], acc_sh.at[fstt_v], add=True)

        plsc.subcore_barrier()

        r0 = sid * 624

        def writeout(cix):
            pltpu.sync_copy(acc_sh.at[pl.ds(r0, 624)],
                            out_hbm.at[cix, pl.ds(r0, 624)])

            @pl.when(sid == _NS - 1)
            def _():
                pltpu.sync_copy(acc_sh.at[pl.ds(9984, 16)],
                                out_hbm.at[cix, pl.ds(9984, 16)])

        @pl.when(cid == 0)
        def _():
            writeout(0)

        @pl.when(cid == 1)
        def _():
            writeout(1)

    return sk


_scatter_call = _build_scatter()


_CB = 1000


def _contrib_body(dist_ref, mu_ref, sig_ref, g_ref, w_ref, c_ref):
    d = dist_ref[0, 0][:, None]
    dinv = 1.0 / d
    mu = mu_ref[0]
    sig = sig_ref[0]
    sen = jnp.exp(-((dinv - mu) ** 2) / (2.0 * sig * sig))
    cut = 0.5 * (jnp.cos(d * (math.pi / DIST_HARD_MAX)) + 1.0)
    cut = jnp.where(d <= DIST_HARD_MAX, cut, 0.0)
    sen = sen * cut
    g = g_ref[...].astype(jnp.bfloat16)
    z = jnp.dot(g, w_ref[...], preferred_element_type=jnp.float32)
    acc = sen[:, 0:1] * z[:, 0:NF]
    for s in range(1, N_SENS):
        acc = acc + sen[:, s:s + 1] * z[:, s * NF:(s + 1) * NF]
    c_ref[...] = acc


def _contrib_tc(dist3, mu3, sig3, G, wcat_bf16):
    grid = (N_PAIRS // _CB,)
    return pl.pallas_call(
        _contrib_body,
        grid=grid,
        in_specs=[
            pl.BlockSpec((1, 1, _CB), lambda i: (i, 0, 0)),
            pl.BlockSpec((1, 1, 32), lambda i: (0, 0, 0)),
            pl.BlockSpec((1, 1, 32), lambda i: (0, 0, 0)),
            pl.BlockSpec((_CB, NF), lambda i: (i, 0)),
            pl.BlockSpec((NF, N_SENS * NF), lambda i: (0, 0)),
        ],
        out_specs=pl.BlockSpec((_CB, NF), lambda i: (i, 0)),
        out_shape=jax.ShapeDtypeStruct((N_PAIRS, NF), jnp.float32),
    )(dist3, mu3, sig3, G, wcat_bf16)


_DR_BLK = 1000


def _softplus(x):
    return jnp.maximum(x, 0.0) + jnp.log1p(jnp.exp(-jnp.abs(x)))


def _dense_body(part_ref, x_ref, swt_ref, sb_ref, iwt_ref, ib_ref,
                awt_ref, ab_ref, rwt_ref, rb_ref, o_ref):
    x = x_ref[...]
    part = part_ref[0] + part_ref[1]
    base = (part
            + jnp.dot(x, swt_ref[...], preferred_element_type=jnp.float32)
            + sb_ref[...])
    t = (jnp.dot(_softplus(base), iwt_ref[...], preferred_element_type=jnp.float32)
         + ib_ref[...] + x)
    for j in range(N_ATOM_LAYERS):
        b2 = jnp.dot(t, awt_ref[j], preferred_element_type=jnp.float32) + ab_ref[j]
        t = (jnp.dot(_softplus(b2), rwt_ref[j], preferred_element_type=jnp.float32)
             + rb_ref[j] + t)
    o_ref[...] = t


def _dense_chain(part, feats, swt, sb, iwt, ib, awt, ab, rwt, rb):
    grid = (N_ATOMS // _DR_BLK,)
    return pl.pallas_call(
        _dense_body,
        grid=grid,
        in_specs=[
            pl.BlockSpec((2, _DR_BLK, NF), lambda i: (0, i, 0)),
            pl.BlockSpec((_DR_BLK, NF), lambda i: (i, 0)),
            pl.BlockSpec((NF, NF), lambda i: (0, 0)),
            pl.BlockSpec((1, NF), lambda i: (0, 0)),
            pl.BlockSpec((NF, NF), lambda i: (0, 0)),
            pl.BlockSpec((1, NF), lambda i: (0, 0)),
            pl.BlockSpec((N_ATOM_LAYERS, NF, NF), lambda i: (0, 0, 0)),
            pl.BlockSpec((N_ATOM_LAYERS, 1, NF), lambda i: (0, 0, 0)),
            pl.BlockSpec((N_ATOM_LAYERS, NF, NF), lambda i: (0, 0, 0)),
            pl.BlockSpec((N_ATOM_LAYERS, 1, NF), lambda i: (0, 0, 0)),
        ],
        out_specs=pl.BlockSpec((_DR_BLK, NF), lambda i: (i, 0)),
        out_shape=jax.ShapeDtypeStruct((N_ATOMS, NF), jnp.float32),
    )(part, feats, swt, sb, iwt, ib, awt, ab, rwt, rb)


def kernel(features, pair_first, pair_second, pair_dist, mu, sigma, int_weights,
           self_W, self_b, intres_W, intres_b, atom_W, atom_b, atomres_W, atomres_b):
    feats = features.astype(jnp.float32)
    dist3 = pair_dist.reshape(N_PAIRS // _CB, 1, _CB)
    pad = jnp.ones((N_BLOCKS, 32 - N_SENS), mu.dtype)
    mu3 = jnp.concatenate([mu, pad], axis=1).reshape(N_BLOCKS, 1, 1, 32)
    sig3 = jnp.concatenate([sigma, pad], axis=1).reshape(N_BLOCKS, 1, 1, 32)
    outs = [feats]
    for b in range(N_BLOCKS):
        wcat = jnp.transpose(int_weights[b], (1, 0, 2)).reshape(NF, N_SENS * NF)
        G = _gather_call(feats, pair_second)
        c = _contrib_tc(dist3, mu3[b], sig3[b], G, wcat.astype(jnp.bfloat16))
        part = _scatter_call(c, pair_first)
        feats = _dense_chain(
            part, feats,
            self_W[b].T, self_b[b].reshape(1, NF),
            intres_W[b].T, intres_b[b].reshape(1, NF),
            jnp.transpose(atom_W[b], (0, 2, 1)), atom_b[b].reshape(N_ATOM_LAYERS, 1, NF),
            jnp.transpose(atomres_W[b], (0, 2, 1)), atomres_b[b].reshape(N_ATOM_LAYERS, 1, NF),
        )
        outs.append(feats)
    return tuple(outs)


# speedup vs baseline: 1.0364x; 1.0364x over previous
import functools
import math

import jax
import jax.numpy as jnp
import numpy as np
from jax import lax
from jax.experimental import pallas as pl
from jax.experimental.pallas import tpu as pltpu
from jax.experimental.pallas import tpu_sc as plsc

N_ATOMS = 10000
N_PAIRS = 320000
NF = 128
N_SENS = 20
N_BLOCKS = 2
N_ATOM_LAYERS = 3
DIST_HARD_MAX = 6.5

_NC = 2
_NS = 16
_NW = _NC * _NS
_K = 128
_PER_W = N_PAIRS // _NW
_NCH = _PER_W // _K
_TAIL = _PER_W - _NCH * _K


def _build_gather():
    mesh = plsc.VectorSubcoreMesh(core_axis_name="c", subcore_axis_name="s")

    @functools.partial(
        pl.kernel,
        mesh=mesh,
        out_type=jax.ShapeDtypeStruct((N_PAIRS, NF), jnp.float32),
        scratch_types=[
            pltpu.VMEM((_PER_W,), jnp.int32),
            pltpu.VMEM((_K, NF), jnp.float32),
            pltpu.VMEM((_K, NF), jnp.float32),
            pltpu.SemaphoreType.DMA,
            pltpu.SemaphoreType.DMA,
        ],
    )
    def gk(feats_hbm, second_hbm, out_hbm, sec_v, rows0_v, rows1_v, sem0, sem1):
        cid = lax.axis_index("c")
        sid = lax.axis_index("s")
        wid = cid * _NS + sid
        base_p = wid * _PER_W
        pltpu.sync_copy(second_hbm.at[pl.ds(base_p, _PER_W)], sec_v)

        def issue(g, rows_v, sem):
            pltpu.async_copy(feats_hbm.at[sec_v.at[pl.ds(g * _K, _K)]], rows_v, sem)

        def drain(rows_v, sem):
            pltpu.make_async_copy(feats_hbm.at[sec_v.at[pl.ds(0, _K)]], rows_v, sem).wait()

        issue(0, rows0_v, sem0)

        def body(h, _):
            g0 = h * 2
            issue(g0 + 1, rows1_v, sem1)
            drain(rows0_v, sem0)
            pltpu.sync_copy(rows0_v, out_hbm.at[pl.ds(base_p + g0 * _K, _K)])

            @pl.when(g0 + 2 < _NCH)
            def _():
                issue(g0 + 2, rows0_v, sem0)

            drain(rows1_v, sem1)
            pltpu.sync_copy(rows1_v, out_hbm.at[pl.ds(base_p + (g0 + 1) * _K, _K)])
            return 0

        lax.fori_loop(0, _NCH // 2, body, 0)
        tp = base_p + _NCH * _K
        pltpu.async_copy(
            feats_hbm.at[sec_v.at[pl.ds(_NCH * _K, _TAIL)]],
            rows0_v.at[pl.ds(0, _TAIL)], sem0).wait()
        pltpu.sync_copy(rows0_v.at[pl.ds(0, _TAIL)], out_hbm.at[pl.ds(tp, _TAIL)])

    return gk


_gather_call = _build_gather()


def _build_scatter():
    mesh = plsc.VectorSubcoreMesh(core_axis_name="c", subcore_axis_name="s")

    @functools.partial(
        pl.kernel,
        mesh=mesh,
        out_type=jax.ShapeDtypeStruct((_NC, N_ATOMS, NF), jnp.float32),
        scratch_types=[
            pltpu.VMEM((_K,), jnp.int32),
            pltpu.VMEM((_K,), jnp.int32),
            pltpu.VMEM((_K, NF), jnp.float32),
            pltpu.VMEM((_K, NF), jnp.float32),
            pltpu.VMEM((_TAIL,), jnp.int32),
            pltpu.VMEM((104, NF), jnp.float32),
            pltpu.VMEM_SHARED((N_ATOMS, NF), jnp.float32),
            pltpu.SemaphoreType.DMA,
            pltpu.SemaphoreType.DMA,
        ],
    )
    def sk(c_hbm, first_hbm, out_hbm, fst0_v, fst1_v, cr0_v, cr1_v, fstt_v,
           zbuf_v, acc_sh, sem0, sem1):
        cid = lax.axis_index("c")
        sid = lax.axis_index("s")
        wid = cid * _NS + sid

        def zfill(i, _):
            zbuf_v[i // 8, pl.ds((i % 8) * 16, 16)] = jnp.zeros((16,), jnp.float32)
            return 0
        lax.fori_loop(0, 104 * 8, zfill, 0)

        def zcopy(i, _):
            pltpu.sync_copy(zbuf_v, acc_sh.at[pl.ds(sid * 624 + i * 104, 104)])
            return 0
        lax.fori_loop(0, 6, zcopy, 0)

        @pl.when(sid == _NS - 1)
        def _():
            pltpu.sync_copy(zbuf_v.at[pl.ds(0, 16)], acc_sh.at[pl.ds(9984, 16)])

        plsc.subcore_barrier()

        base_p = wid * _PER_W

        def load(g, fst_v, cr_v, sem):
            p0 = base_p + g * _K
            pltpu.async_copy(first_hbm.at[pl.ds(p0, _K)], fst_v, sem)
            pltpu.async_copy(c_hbm.at[pl.ds(p0, _K)], cr_v, sem)

        def drain(fst_v, cr_v, sem):
            pltpu.make_async_copy(first_hbm.at[pl.ds(0, _K)], fst_v, sem).wait()
            pltpu.make_async_copy(c_hbm.at[pl.ds(0, _K)], cr_v, sem).wait()

        def scat(fst_v, cr_v):
            pltpu.sync_copy(cr_v, acc_sh.at[fst_v], add=True)

        load(0, fst0_v, cr0_v, sem0)

        def body(h, _):
            g0 = h * 2
            load(g0 + 1, fst1_v, cr1_v, sem1)
            drain(fst0_v, cr0_v, sem0)
            scat(fst0_v, cr0_v)

            @pl.when(g0 + 2 < _NCH)
            def _():
                load(g0 + 2, fst0_v, cr0_v, sem0)

            drain(fst1_v, cr1_v, sem1)
            scat(fst1_v, cr1_v)
            return 0

        lax.fori_loop(0, _NCH // 2, body, 0)

        tp = base_p + _NCH * _K
        pltpu.sync_copy(first_hbm.at[pl.ds(tp, _TAIL)], fstt_v)
        pltpu.async_copy(c_hbm.at[pl.ds(tp, _TAIL)],
                         cr0_v.at[pl.ds(0, _TAIL)], sem0).wait()
        pltpu.sync_copy(cr0_v.at[pl.ds(0, _TAIL)], acc_sh.at[fstt_v], add=True)

        plsc.subcore_barrier()

        r0 = sid * 624

        def writeout(cix):
            pltpu.sync_copy(acc_sh.at[pl.ds(r0, 624)],
                            out_hbm.at[cix, pl.ds(r0, 624)])

            @pl.when(sid == _NS - 1)
            def _():
                pltpu.sync_copy(acc_sh.at[pl.ds(9984, 16)],
                                out_hbm.at[cix, pl.ds(9984, 16)])

        @pl.when(cid == 0)
        def _():
            writeout(0)

        @pl.when(cid == 1)
        def _():
            writeout(1)

    return sk


_scatter_call = _build_scatter()


_CB = 2000


def _contrib_body(dist_ref, mu_ref, sig_ref, g_ref, w_ref, c_ref):
    d = dist_ref[0, 0][:, None]
    dinv = 1.0 / d
    mu = mu_ref[0]
    sig = sig_ref[0]
    sen = jnp.exp(-((dinv - mu) ** 2) / (2.0 * sig * sig))
    cut = 0.5 * (jnp.cos(d * (math.pi / DIST_HARD_MAX)) + 1.0)
    cut = jnp.where(d <= DIST_HARD_MAX, cut, 0.0)
    sen = sen * cut
    g = g_ref[...].astype(jnp.bfloat16)
    z = jnp.dot(g, w_ref[...], preferred_element_type=jnp.float32)
    acc = sen[:, 0:1] * z[:, 0:NF]
    for s in range(1, N_SENS):
        acc = acc + sen[:, s:s + 1] * z[:, s * NF:(s + 1) * NF]
    c_ref[...] = acc


def _contrib_tc(dist3, mu3, sig3, G, wcat_bf16):
    grid = (N_PAIRS // _CB,)
    return pl.pallas_call(
        _contrib_body,
        grid=grid,
        in_specs=[
            pl.BlockSpec((1, 1, _CB), lambda i: (i, 0, 0)),
            pl.BlockSpec((1, 1, 32), lambda i: (0, 0, 0)),
            pl.BlockSpec((1, 1, 32), lambda i: (0, 0, 0)),
            pl.BlockSpec((_CB, NF), lambda i: (i, 0)),
            pl.BlockSpec((NF, N_SENS * NF), lambda i: (0, 0)),
        ],
        out_specs=pl.BlockSpec((_CB, NF), lambda i: (i, 0)),
        out_shape=jax.ShapeDtypeStruct((N_PAIRS, NF), jnp.float32),
    )(dist3, mu3, sig3, G, wcat_bf16)


_DR_BLK = 1000


def _softplus(x):
    return jnp.maximum(x, 0.0) + jnp.log1p(jnp.exp(-jnp.abs(x)))


def _dense_body(part_ref, x_ref, swt_ref, sb_ref, iwt_ref, ib_ref,
                awt_ref, ab_ref, rwt_ref, rb_ref, o_ref):
    x = x_ref[...]
    part = part_ref[0] + part_ref[1]
    base = (part
            + jnp.dot(x, swt_ref[...], preferred_element_type=jnp.float32)
            + sb_ref[...])
    t = (jnp.dot(_softplus(base), iwt_ref[...], preferred_element_type=jnp.float32)
         + ib_ref[...] + x)
    for j in range(N_ATOM_LAYERS):
        b2 = jnp.dot(t, awt_ref[j], preferred_element_type=jnp.float32) + ab_ref[j]
        t = (jnp.dot(_softplus(b2), rwt_ref[j], preferred_element_type=jnp.float32)
             + rb_ref[j] + t)
    o_ref[...] = t


def _dense_chain(part, feats, swt, sb, iwt, ib, awt, ab, rwt, rb):
    grid = (N_ATOMS // _DR_BLK,)
    return pl.pallas_call(
        _dense_body,
        grid=grid,
        in_specs=[
            pl.BlockSpec((2, _DR_BLK, NF), lambda i: (0, i, 0)),
            pl.BlockSpec((_DR_BLK, NF), lambda i: (i, 0)),
            pl.BlockSpec((NF, NF), lambda i: (0, 0)),
            pl.BlockSpec((1, NF), lambda i: (0, 0)),
            pl.BlockSpec((NF, NF), lambda i: (0, 0)),
            pl.BlockSpec((1, NF), lambda i: (0, 0)),
            pl.BlockSpec((N_ATOM_LAYERS, NF, NF), lambda i: (0, 0, 0)),
            pl.BlockSpec((N_ATOM_LAYERS, 1, NF), lambda i: (0, 0, 0)),
            pl.BlockSpec((N_ATOM_LAYERS, NF, NF), lambda i: (0, 0, 0)),
            pl.BlockSpec((N_ATOM_LAYERS, 1, NF), lambda i: (0, 0, 0)),
        ],
        out_specs=pl.BlockSpec((_DR_BLK, NF), lambda i: (i, 0)),
        out_shape=jax.ShapeDtypeStruct((N_ATOMS, NF), jnp.float32),
    )(part, feats, swt, sb, iwt, ib, awt, ab, rwt, rb)


def kernel(features, pair_first, pair_second, pair_dist, mu, sigma, int_weights,
           self_W, self_b, intres_W, intres_b, atom_W, atom_b, atomres_W, atomres_b):
    feats = features.astype(jnp.float32)
    dist3 = pair_dist.reshape(N_PAIRS // _CB, 1, _CB)
    pad = jnp.ones((N_BLOCKS, 32 - N_SENS), mu.dtype)
    mu3 = jnp.concatenate([mu, pad], axis=1).reshape(N_BLOCKS, 1, 1, 32)
    sig3 = jnp.concatenate([sigma, pad], axis=1).reshape(N_BLOCKS, 1, 1, 32)
    outs = [feats]
    for b in range(N_BLOCKS):
        wcat = jnp.transpose(int_weights[b], (1, 0, 2)).reshape(NF, N_SENS * NF)
        G = _gather_call(feats, pair_second)
        c = _contrib_tc(dist3, mu3[b], sig3[b], G, wcat.astype(jnp.bfloat16))
        part = _scatter_call(c, pair_first)
        feats = _dense_chain(
            part, feats,
            self_W[b].T, self_b[b].reshape(1, NF),
            intres_W[b].T, intres_b[b].reshape(1, NF),
            jnp.transpose(atom_W[b], (0, 2, 1)), atom_b[b].reshape(N_ATOM_LAYERS, 1, NF),
            jnp.transpose(atomres_W[b], (0, 2, 1)), atomres_b[b].reshape(N_ATOM_LAYERS, 1, NF),
        )
        outs.append(feats)
    return tuple(outs)
